# baseline (speedup 1.0000x reference)
import functools

import jax
import jax.numpy as jnp
from jax import lax
from jax.experimental import pallas as pl
from jax.experimental.pallas import tpu as pltpu
from jax.experimental.pallas import tpu_sc as plsc

N = 10000
E = 160000
D = 256
H = 512

LANES = 128
LB = 128
NPAD = 10240
ROWS = NPAD // LANES
QB = H // LB
BLK = 1024

NC = 2
NS = 16
CHE = 80
EPAD = NS * CHE * LANES
SLICE = NPAD // NS
RING = 1


def _mlp_body(x_ref, w1_ref, b1_ref, h_ref):
    h = jnp.dot(x_ref[...], w1_ref[...], preferred_element_type=jnp.float32)
    h = jnp.maximum(h + b1_ref[...], 0.0)
    for q in range(QB):
        h_ref[q] = h[:, q * LB:(q + 1) * LB]


def _hidden(x_pad, W1, b1):
    return pl.pallas_call(
        _mlp_body,
        grid=(NPAD // BLK,),
        in_specs=[
            pl.BlockSpec((BLK, D), lambda i: (i, 0)),
            pl.BlockSpec((D, H), lambda i: (0, 0)),
            pl.BlockSpec((1, H), lambda i: (0, 0)),
        ],
        out_specs=pl.BlockSpec((QB, BLK, LB), lambda i: (0, i, 0)),
        out_shape=jax.ShapeDtypeStruct((QB, NPAD, LB), jnp.float32),
    )(x_pad, W1, b1.reshape(1, H))


def _segsum_body(h_hbm, src_hbm, dst_hbm, zeros_hbm, out_hbm,
                 src_v, dst_v, buf, acc_sh, sem):
    cid = lax.axis_index("c")
    sid = lax.axis_index("s")

    pltpu.sync_copy(src_hbm.at[sid], src_v)
    pltpu.sync_copy(dst_hbm.at[sid], dst_v)

    for k in range(QB // NC):
        q = cid * (QB // NC) + k

        pltpu.sync_copy(zeros_hbm.at[pl.ds(sid * SLICE, SLICE)],
                        acc_sh.at[pl.ds(sid * SLICE, SLICE)])
        plsc.subcore_barrier()

        def step(j, carry):
            pltpu.async_copy(
                h_hbm.at[q].at[src_v.at[j]], buf, sem).wait()
            pltpu.sync_copy(buf, acc_sh.at[dst_v.at[j]], add=True)
            return carry

        lax.fori_loop(0, CHE, step, 0)
        plsc.subcore_barrier()

        pltpu.sync_copy(acc_sh.at[pl.ds(sid * SLICE, SLICE)],
                        out_hbm.at[q, pl.ds(sid * SLICE, SLICE)])
        plsc.subcore_barrier()


@functools.cache
def _segsum():
    return pl.kernel(
        _segsum_body,
        out_type=jax.ShapeDtypeStruct((QB, NPAD, LB), jnp.float32),
        mesh=plsc.VectorSubcoreMesh(core_axis_name="c", subcore_axis_name="s",
                                    num_cores=NC, num_subcores=NS),
        scratch_types=(
            [pltpu.VMEM((CHE, LANES), jnp.int32),
             pltpu.VMEM((CHE, LANES), jnp.int32)]
            + [pltpu.VMEM((LANES, LB), jnp.float32)]
            + [pltpu.VMEM_SHARED((NPAD, LB), jnp.float32)]
            + [pltpu.SemaphoreType.DMA]
        ),
    )


def _pred_body(h_ref, a_ref, w2_ref, b2_ref, o_ref):
    a = jnp.concatenate([h_ref[q] + a_ref[q] for q in range(QB)], axis=1)
    o_ref[...] = jnp.dot(a, w2_ref[...],
                         preferred_element_type=jnp.float32) + b2_ref[0, 0]


def _pred(h_blk, agg_blk, W2, b2):
    return pl.pallas_call(
        _pred_body,
        grid=(NPAD // BLK,),
        in_specs=[
            pl.BlockSpec((QB, BLK, LB), lambda i: (0, i, 0)),
            pl.BlockSpec((QB, BLK, LB), lambda i: (0, i, 0)),
            pl.BlockSpec((H, 1), lambda i: (0, 0)),
            pl.BlockSpec((1, 1), lambda i: (0, 0)),
        ],
        out_specs=pl.BlockSpec((BLK, 1), lambda i: (i, 0)),
        out_shape=jax.ShapeDtypeStruct((NPAD, 1), jnp.float32),
    )(h_blk, agg_blk, W2, b2.reshape(1, 1))


def _finalize_body(pred_ref, g_ref, act_ref, lp_ref):
    pred = pred_ref[...]
    row = lax.broadcasted_iota(jnp.int32, (ROWS, LANES), 0)
    col = lax.broadcasted_iota(jnp.int32, (ROWS, LANES), 1)
    flat = row * LANES + col
    valid = flat < N
    neg_inf = jnp.float32(-jnp.inf)

    m = jnp.max(jnp.where(valid, pred, neg_inf))
    e = jnp.where(valid, jnp.exp(pred - m), 0.0)
    p = e / jnp.sum(e)

    y = jnp.where(valid, jnp.log(p + 1e-20) + g_ref[...], neg_inf)
    ymax = jnp.max(y)
    action = jnp.min(jnp.where(y == ymax, flat, jnp.int32(2147483647)))
    act_ref[0, 0] = action
    lp_ref[0, 0] = jnp.log(jnp.sum(jnp.where(flat == action, p, 0.0)))


def _finalize(pred80, g80):
    return pl.pallas_call(
        _finalize_body,
        in_specs=[
            pl.BlockSpec(memory_space=pltpu.VMEM),
            pl.BlockSpec(memory_space=pltpu.VMEM),
        ],
        out_specs=[
            pl.BlockSpec(memory_space=pltpu.SMEM),
            pl.BlockSpec(memory_space=pltpu.SMEM),
        ],
        out_shape=[
            jax.ShapeDtypeStruct((1, 1), jnp.int32),
            jax.ShapeDtypeStruct((1, 1), jnp.float32),
        ],
    )(pred80, g80)


def kernel(x, edge_index, batch, W1, b1, W2, b2):
    x_pad = jnp.concatenate([x, jnp.zeros((NPAD - N, D), jnp.float32)])
    h_blk = _hidden(x_pad, W1, b1)

    src = jnp.concatenate(
        [edge_index[0], jnp.zeros((EPAD - E,), jnp.int32)]).reshape(NS, CHE, LANES)
    dst = jnp.concatenate(
        [edge_index[1], jnp.full((EPAD - E,), N, jnp.int32)]).reshape(NS, CHE, LANES)
    zeros = jnp.zeros((NPAD, LB), jnp.float32)

    agg_blk = _segsum()(h_blk, src, dst, zeros)

    pred_pad = _pred(h_blk, agg_blk, W2, b2)

    g = jax.random.gumbel(jax.random.key(42), (N,), jnp.float32)
    g80 = jnp.concatenate([g, jnp.zeros((NPAD - N,), jnp.float32)]
                          ).reshape(ROWS, LANES)

    act, lp = _finalize(pred_pad.reshape(ROWS, LANES), g80)
    return (pred_pad[:N], act.reshape(()), lp.reshape(()))


# speedup vs baseline: 1.3333x; 1.3333x over previous
import functools

import jax
import jax.numpy as jnp
from jax import lax
from jax.experimental import pallas as pl
from jax.experimental.pallas import tpu as pltpu
from jax.experimental.pallas import tpu_sc as plsc

N = 10000
E = 160000
D = 256
H = 512

LANES = 128
LB = 128
NPAD = 10240
ROWS = NPAD // LANES
QB = H // LB
BLK = 1024

NC = 2
NS = 16
CHE = 79
EPAD = NS * CHE * LANES
SLICE = NPAD // NS
RING = 1


def _mlp_body(x_ref, w1_ref, b1_ref, h_ref):
    h = jnp.dot(x_ref[...], w1_ref[...], preferred_element_type=jnp.float32)
    h = jnp.maximum(h + b1_ref[...], 0.0)
    for q in range(QB):
        h_ref[q] = h[:, q * LB:(q + 1) * LB]


def _hidden(x_pad, W1, b1):
    return pl.pallas_call(
        _mlp_body,
        grid=(NPAD // BLK,),
        in_specs=[
            pl.BlockSpec((BLK, D), lambda i: (i, 0)),
            pl.BlockSpec((D, H), lambda i: (0, 0)),
            pl.BlockSpec((1, H), lambda i: (0, 0)),
        ],
        out_specs=pl.BlockSpec((QB, BLK, LB), lambda i: (0, i, 0)),
        out_shape=jax.ShapeDtypeStruct((QB, NPAD, LB), jnp.float32),
    )(x_pad, W1, b1.reshape(1, H))


def _segsum_body(h_hbm, src_hbm, dst_hbm, zeros_hbm, out_hbm,
                 src_v, dst_v, buf, acc_sh, sem):
    cid = lax.axis_index("c")
    sid = lax.axis_index("s")

    pltpu.sync_copy(src_hbm.at[sid], src_v)
    pltpu.sync_copy(dst_hbm.at[sid], dst_v)

    for k in range(QB // NC):
        q = cid * (QB // NC) + k

        pltpu.sync_copy(zeros_hbm.at[pl.ds(sid * SLICE, SLICE)],
                        acc_sh.at[pl.ds(sid * SLICE, SLICE)])
        plsc.subcore_barrier()

        def step(j, carry):
            pltpu.async_copy(
                h_hbm.at[q].at[src_v.at[j]], buf, sem).wait()
            pltpu.sync_copy(buf, acc_sh.at[dst_v.at[j]], add=True)
            return carry

        lax.fori_loop(0, CHE, step, 0)
        plsc.subcore_barrier()

        pltpu.sync_copy(acc_sh.at[pl.ds(sid * SLICE, SLICE)],
                        out_hbm.at[q, pl.ds(sid * SLICE, SLICE)])
        plsc.subcore_barrier()


@functools.cache
def _segsum():
    return pl.kernel(
        _segsum_body,
        out_type=jax.ShapeDtypeStruct((QB, NPAD, LB), jnp.float32),
        mesh=plsc.VectorSubcoreMesh(core_axis_name="c", subcore_axis_name="s",
                                    num_cores=NC, num_subcores=NS),
        scratch_types=(
            [pltpu.VMEM((CHE, LANES), jnp.int32),
             pltpu.VMEM((CHE, LANES), jnp.int32)]
            + [pltpu.VMEM((LANES, LB), jnp.float32)]
            + [pltpu.VMEM_SHARED((NPAD, LB), jnp.float32)]
            + [pltpu.SemaphoreType.DMA]
        ),
    )


def _pred_body(h_ref, a_ref, w2_ref, b2_ref, o_ref):
    a = jnp.concatenate([h_ref[q] + a_ref[q] for q in range(QB)], axis=1)
    o_ref[...] = jnp.dot(a, w2_ref[...],
                         preferred_element_type=jnp.float32) + b2_ref[0, 0]


def _pred(h_blk, agg_blk, W2, b2):
    return pl.pallas_call(
        _pred_body,
        grid=(NPAD // BLK,),
        in_specs=[
            pl.BlockSpec((QB, BLK, LB), lambda i: (0, i, 0)),
            pl.BlockSpec((QB, BLK, LB), lambda i: (0, i, 0)),
            pl.BlockSpec((H, 1), lambda i: (0, 0)),
            pl.BlockSpec((1, 1), lambda i: (0, 0)),
        ],
        out_specs=pl.BlockSpec((BLK, 1), lambda i: (i, 0)),
        out_shape=jax.ShapeDtypeStruct((NPAD, 1), jnp.float32),
    )(h_blk, agg_blk, W2, b2.reshape(1, 1))


def _finalize_body(pred_ref, g_ref, act_ref, lp_ref):
    pred = pred_ref[...]
    row = lax.broadcasted_iota(jnp.int32, (ROWS, LANES), 0)
    col = lax.broadcasted_iota(jnp.int32, (ROWS, LANES), 1)
    flat = row * LANES + col
    valid = flat < N
    neg_inf = jnp.float32(-jnp.inf)

    m = jnp.max(jnp.where(valid, pred, neg_inf))
    e = jnp.where(valid, jnp.exp(pred - m), 0.0)
    p = e / jnp.sum(e)

    y = jnp.where(valid, jnp.log(p + 1e-20) + g_ref[...], neg_inf)
    ymax = jnp.max(y)
    action = jnp.min(jnp.where(y == ymax, flat, jnp.int32(2147483647)))
    act_ref[0, 0] = action
    lp_ref[0, 0] = jnp.log(jnp.sum(jnp.where(flat == action, p, 0.0)))


def _finalize(pred80, g80):
    return pl.pallas_call(
        _finalize_body,
        in_specs=[
            pl.BlockSpec(memory_space=pltpu.VMEM),
            pl.BlockSpec(memory_space=pltpu.VMEM),
        ],
        out_specs=[
            pl.BlockSpec(memory_space=pltpu.SMEM),
            pl.BlockSpec(memory_space=pltpu.SMEM),
        ],
        out_shape=[
            jax.ShapeDtypeStruct((1, 1), jnp.int32),
            jax.ShapeDtypeStruct((1, 1), jnp.float32),
        ],
    )(pred80, g80)


def kernel(x, edge_index, batch, W1, b1, W2, b2):
    x_pad = jnp.concatenate([x, jnp.zeros((NPAD - N, D), jnp.float32)])
    h_blk = _hidden(x_pad, W1, b1)

    src = jnp.concatenate(
        [edge_index[0], jnp.zeros((EPAD - E,), jnp.int32)]).reshape(NS, CHE, LANES)
    dst = jnp.concatenate(
        [edge_index[1], jnp.full((EPAD - E,), N, jnp.int32)]).reshape(NS, CHE, LANES)
    zeros = jnp.zeros((NPAD, LB), jnp.float32)

    agg_blk = _segsum()(h_blk, src, dst, zeros)

    pred_pad = _pred(h_blk, agg_blk, W2, b2)

    g = jax.random.gumbel(jax.random.key(42), (N,), jnp.float32)
    g80 = jnp.concatenate([g, jnp.zeros((NPAD - N,), jnp.float32)]
                          ).reshape(ROWS, LANES)

    act, lp = _finalize(pred_pad.reshape(ROWS, LANES), g80)
    return (pred_pad[:N], act.reshape(()), lp.reshape(()))


# speedup vs baseline: 1.3493x; 1.0120x over previous
import functools

import jax
import jax.numpy as jnp
from jax import lax
from jax.experimental import pallas as pl
from jax.experimental.pallas import tpu as pltpu
from jax.experimental.pallas import tpu_sc as plsc

N = 10000
E = 160000
D = 256
H = 512

LANES = 128
LB = 128
NPAD = 10240
ROWS = NPAD // LANES
QB = H // LB
BLK = 1024

NC = 2
NS = 16
CHE = 79
EPAD = NS * CHE * LANES
SLICE = NPAD // NS
RING = 1


def _mlp_body(x_ref, w1_ref, b1_ref, h_ref):
    h = jnp.dot(x_ref[...], w1_ref[...], preferred_element_type=jnp.float32)
    h = jnp.maximum(h + b1_ref[...], 0.0)
    for q in range(QB):
        h_ref[q] = h[:, q * LB:(q + 1) * LB]


def _hidden(x_pad, W1, b1):
    return pl.pallas_call(
        _mlp_body,
        grid=(NPAD // BLK,),
        in_specs=[
            pl.BlockSpec((BLK, D), lambda i: (i, 0)),
            pl.BlockSpec((D, H), lambda i: (0, 0)),
            pl.BlockSpec((1, H), lambda i: (0, 0)),
        ],
        out_specs=pl.BlockSpec((QB, BLK, LB), lambda i: (0, i, 0)),
        out_shape=jax.ShapeDtypeStruct((QB, NPAD, LB), jnp.float32),
    )(x_pad, W1, b1.reshape(1, H))


def _segsum_body(h_hbm, src_hbm, dst_hbm, zeros_hbm, out_hbm,
                 src_v, dst_v, buf, acc_sh, sem):
    cid = lax.axis_index("c")
    sid = lax.axis_index("s")

    pltpu.sync_copy(src_hbm.at[sid], src_v)
    pltpu.sync_copy(dst_hbm.at[sid], dst_v)

    for k in range(QB // NC):
        q = cid * (QB // NC) + k

        pltpu.sync_copy(zeros_hbm.at[pl.ds(sid * SLICE, SLICE)],
                        acc_sh.at[pl.ds(sid * SLICE, SLICE)])
        plsc.subcore_barrier()

        def step(j, carry):
            pltpu.async_copy(
                h_hbm.at[q].at[src_v.at[j]], buf, sem).wait()
            pltpu.sync_copy(buf, acc_sh.at[dst_v.at[j]], add=True)
            return carry

        lax.fori_loop(0, CHE, step, 0)
        plsc.subcore_barrier()

        pltpu.sync_copy(acc_sh.at[pl.ds(sid * SLICE, SLICE)],
                        out_hbm.at[q, pl.ds(sid * SLICE, SLICE)])
        plsc.subcore_barrier()


@functools.cache
def _segsum():
    return pl.kernel(
        _segsum_body,
        out_type=jax.ShapeDtypeStruct((QB, NPAD, LB), jnp.float32),
        mesh=plsc.VectorSubcoreMesh(core_axis_name="c", subcore_axis_name="s",
                                    num_cores=NC, num_subcores=NS),
        scratch_types=(
            [pltpu.VMEM((CHE, LANES), jnp.int32),
             pltpu.VMEM((CHE, LANES), jnp.int32)]
            + [pltpu.VMEM((LANES, LB), jnp.float32)]
            + [pltpu.VMEM_SHARED((NPAD, LB), jnp.float32)]
            + [pltpu.SemaphoreType.DMA]
        ),
    )


def _pred_body(h_ref, a_ref, w2_ref, b2_ref, g_ref,
               pred_ref, act_ref, lp_ref, scratch):
    i = pl.program_id(0)
    a = jnp.concatenate([h_ref[q] + a_ref[q] for q in range(QB)], axis=1)
    pred_blk = jnp.dot(a, w2_ref[...],
                       preferred_element_type=jnp.float32) + b2_ref[0, 0]
    pred_ref[...] = pred_blk
    scratch[pl.ds(i * (BLK // LANES), BLK // LANES), :] = (
        pred_blk.reshape(BLK // LANES, LANES))

    @pl.when(i == NPAD // BLK - 1)
    def _():
        pred = scratch[...]
        row = lax.broadcasted_iota(jnp.int32, (ROWS, LANES), 0)
        col = lax.broadcasted_iota(jnp.int32, (ROWS, LANES), 1)
        flat = row * LANES + col
        valid = flat < N
        neg_inf = jnp.float32(-jnp.inf)

        m = jnp.max(jnp.where(valid, pred, neg_inf))
        e = jnp.where(valid, jnp.exp(pred - m), 0.0)
        p = e / jnp.sum(e)

        y = jnp.where(valid, jnp.log(p + 1e-20) + g_ref[...], neg_inf)
        ymax = jnp.max(y)
        action = jnp.min(jnp.where(y == ymax, flat, jnp.int32(2147483647)))
        act_ref[0, 0] = action
        lp_ref[0, 0] = jnp.log(jnp.sum(jnp.where(flat == action, p, 0.0)))


def _pred_sample(h_blk, agg_blk, W2, b2, g80):
    return pl.pallas_call(
        _pred_body,
        grid=(NPAD // BLK,),
        in_specs=[
            pl.BlockSpec((QB, BLK, LB), lambda i: (0, i, 0)),
            pl.BlockSpec((QB, BLK, LB), lambda i: (0, i, 0)),
            pl.BlockSpec((H, 1), lambda i: (0, 0)),
            pl.BlockSpec((1, 1), lambda i: (0, 0)),
            pl.BlockSpec((ROWS, LANES), lambda i: (0, 0)),
        ],
        out_specs=[
            pl.BlockSpec((BLK, 1), lambda i: (i, 0)),
            pl.BlockSpec((1, 1), lambda i: (0, 0), memory_space=pltpu.SMEM),
            pl.BlockSpec((1, 1), lambda i: (0, 0), memory_space=pltpu.SMEM),
        ],
        out_shape=[
            jax.ShapeDtypeStruct((NPAD, 1), jnp.float32),
            jax.ShapeDtypeStruct((1, 1), jnp.int32),
            jax.ShapeDtypeStruct((1, 1), jnp.float32),
        ],
        scratch_shapes=[pltpu.VMEM((ROWS, LANES), jnp.float32)],
    )(h_blk, agg_blk, W2, b2.reshape(1, 1), g80)


def kernel(x, edge_index, batch, W1, b1, W2, b2):
    x_pad = jnp.concatenate([x, jnp.zeros((NPAD - N, D), jnp.float32)])
    h_blk = _hidden(x_pad, W1, b1)

    src = jnp.concatenate(
        [edge_index[0], jnp.zeros((EPAD - E,), jnp.int32)]).reshape(NS, CHE, LANES)
    dst = jnp.concatenate(
        [edge_index[1], jnp.full((EPAD - E,), N, jnp.int32)]).reshape(NS, CHE, LANES)
    zeros = jnp.zeros((NPAD, LB), jnp.float32)

    agg_blk = _segsum()(h_blk, src, dst, zeros)

    g = jax.random.gumbel(jax.random.key(42), (N,), jnp.float32)
    g80 = jnp.concatenate([g, jnp.zeros((NPAD - N,), jnp.float32)]
                          ).reshape(ROWS, LANES)

    pred_pad, act, lp = _pred_sample(h_blk, agg_blk, W2, b2, g80)
    return (pred_pad[:N], act.reshape(()), lp.reshape(()))


# speedup vs baseline: 1.3532x; 1.0029x over previous
import functools

import jax
import jax.numpy as jnp
from jax import lax
from jax.experimental import pallas as pl
from jax.experimental.pallas import tpu as pltpu
from jax.experimental.pallas import tpu_sc as plsc

N = 10000
E = 160000
D = 256
H = 512

LANES = 128
LB = 128
NPAD = 10240
ROWS = NPAD // LANES
QB = H // LB
BLK = 1024

NC = 2
NS = 16
CHE = 79
EPAD = NS * CHE * LANES
SLICE = NPAD // NS
RING = 1


def _mlp_body(x_ref, w1_ref, b1_ref, h_ref):
    h = jnp.dot(x_ref[...], w1_ref[...], preferred_element_type=jnp.float32)
    h = jnp.maximum(h + b1_ref[...], 0.0)
    for q in range(QB):
        h_ref[q] = h[:, q * LB:(q + 1) * LB]


def _hidden(x, W1, b1):
    return pl.pallas_call(
        _mlp_body,
        grid=(NPAD // BLK,),
        in_specs=[
            pl.BlockSpec((BLK, D), lambda i: (i, 0)),
            pl.BlockSpec((D, H), lambda i: (0, 0)),
            pl.BlockSpec((1, H), lambda i: (0, 0)),
        ],
        out_specs=pl.BlockSpec((QB, BLK, LB), lambda i: (0, i, 0)),
        out_shape=jax.ShapeDtypeStruct((QB, NPAD, LB), jnp.float32),
    )(x, W1, b1.reshape(1, H))


def _segsum_body(h_hbm, src_hbm, dst_hbm, zeros_hbm, out_hbm,
                 src_v, dst_v, buf, acc_sh, sem):
    cid = lax.axis_index("c")
    sid = lax.axis_index("s")

    pltpu.sync_copy(src_hbm.at[sid], src_v)
    pltpu.sync_copy(dst_hbm.at[sid], dst_v)

    for k in range(QB // NC):
        q = cid * (QB // NC) + k

        pltpu.sync_copy(zeros_hbm.at[pl.ds(sid * SLICE, SLICE)],
                        acc_sh.at[pl.ds(sid * SLICE, SLICE)])
        plsc.subcore_barrier()

        def step(j, carry):
            pltpu.async_copy(
                h_hbm.at[q].at[src_v.at[j]], buf, sem).wait()
            pltpu.sync_copy(buf, acc_sh.at[dst_v.at[j]], add=True)
            return carry

        lax.fori_loop(0, CHE, step, 0)
        plsc.subcore_barrier()

        pltpu.sync_copy(acc_sh.at[pl.ds(sid * SLICE, SLICE)],
                        out_hbm.at[q, pl.ds(sid * SLICE, SLICE)])
        plsc.subcore_barrier()


@functools.cache
def _segsum():
    return pl.kernel(
        _segsum_body,
        out_type=jax.ShapeDtypeStruct((QB, NPAD, LB), jnp.float32),
        mesh=plsc.VectorSubcoreMesh(core_axis_name="c", subcore_axis_name="s",
                                    num_cores=NC, num_subcores=NS),
        scratch_types=(
            [pltpu.VMEM((CHE, LANES), jnp.int32),
             pltpu.VMEM((CHE, LANES), jnp.int32)]
            + [pltpu.VMEM((LANES, LB), jnp.float32)]
            + [pltpu.VMEM_SHARED((NPAD, LB), jnp.float32)]
            + [pltpu.SemaphoreType.DMA]
        ),
    )


def _pred_body(h_ref, a_ref, w2_ref, b2_ref, g_ref,
               pred_ref, act_ref, lp_ref, scratch):
    i = pl.program_id(0)
    a = jnp.concatenate([h_ref[q] + a_ref[q] for q in range(QB)], axis=1)
    pred_blk = jnp.dot(a, w2_ref[...],
                       preferred_element_type=jnp.float32) + b2_ref[0, 0]
    pred_ref[...] = pred_blk
    scratch[pl.ds(i * (BLK // LANES), BLK // LANES), :] = (
        pred_blk.reshape(BLK // LANES, LANES))

    @pl.when(i == NPAD // BLK - 1)
    def _():
        pred = scratch[...]
        row = lax.broadcasted_iota(jnp.int32, (ROWS, LANES), 0)
        col = lax.broadcasted_iota(jnp.int32, (ROWS, LANES), 1)
        flat = row * LANES + col
        valid = flat < N
        neg_inf = jnp.float32(-jnp.inf)

        m = jnp.max(jnp.where(valid, pred, neg_inf))
        e = jnp.where(valid, jnp.exp(pred - m), 0.0)
        p = e / jnp.sum(e)

        y = jnp.where(valid, jnp.log(p + 1e-20) + g_ref[...], neg_inf)
        ymax = jnp.max(y)
        action = jnp.min(jnp.where(y == ymax, flat, jnp.int32(2147483647)))
        act_ref[0, 0] = action
        lp_ref[0, 0] = jnp.log(jnp.sum(jnp.where(flat == action, p, 0.0)))


def _pred_sample(h_blk, agg_blk, W2, b2, g80):
    return pl.pallas_call(
        _pred_body,
        grid=(NPAD // BLK,),
        in_specs=[
            pl.BlockSpec((QB, BLK, LB), lambda i: (0, i, 0)),
            pl.BlockSpec((QB, BLK, LB), lambda i: (0, i, 0)),
            pl.BlockSpec((H, 1), lambda i: (0, 0)),
            pl.BlockSpec((1, 1), lambda i: (0, 0)),
            pl.BlockSpec((ROWS, LANES), lambda i: (0, 0)),
        ],
        out_specs=[
            pl.BlockSpec((BLK, 1), lambda i: (i, 0)),
            pl.BlockSpec((1, 1), lambda i: (0, 0), memory_space=pltpu.SMEM),
            pl.BlockSpec((1, 1), lambda i: (0, 0), memory_space=pltpu.SMEM),
        ],
        out_shape=[
            jax.ShapeDtypeStruct((NPAD, 1), jnp.float32),
            jax.ShapeDtypeStruct((1, 1), jnp.int32),
            jax.ShapeDtypeStruct((1, 1), jnp.float32),
        ],
        scratch_shapes=[pltpu.VMEM((ROWS, LANES), jnp.float32)],
    )(h_blk, agg_blk, W2, b2.reshape(1, 1), g80)


def kernel(x, edge_index, batch, W1, b1, W2, b2):
    h_blk = _hidden(x, W1, b1)

    src = jnp.concatenate(
        [edge_index[0], jnp.zeros((EPAD - E,), jnp.int32)]).reshape(NS, CHE, LANES)
    dst = jnp.concatenate(
        [edge_index[1], jnp.full((EPAD - E,), N, jnp.int32)]).reshape(NS, CHE, LANES)
    zeros = jnp.zeros((NPAD, LB), jnp.float32)

    agg_blk = _segsum()(h_blk, src, dst, zeros)

    g = jax.random.gumbel(jax.random.key(42), (N,), jnp.float32)
    g80 = jnp.concatenate([g, jnp.zeros((NPAD - N,), jnp.float32)]
                          ).reshape(ROWS, LANES)

    pred_pad, act, lp = _pred_sample(h_blk, agg_blk, W2, b2, g80)
    return (pred_pad[:N], act.reshape(()), lp.reshape(()))


# speedup vs baseline: 1.3552x; 1.0015x over previous
import functools

import jax
import jax.numpy as jnp
from jax import lax
from jax.experimental import pallas as pl
from jax.experimental.pallas import tpu as pltpu
from jax.experimental.pallas import tpu_sc as plsc

N = 10000
E = 160000
D = 256
H = 512

LANES = 128
LB = 128
NPAD = 10240
ROWS = NPAD // LANES
QB = H // LB
BLK = 1024

NC = 2
NS = 16
CHE = 79
EPAD = NS * CHE * LANES
SLICE = NPAD // NS
RING = 1


def _mlp_body(x_ref, w1_ref, b1_ref, h_ref):
    h = jnp.dot(x_ref[...], w1_ref[...], preferred_element_type=jnp.float32)
    h = jnp.maximum(h + b1_ref[...], 0.0)
    for q in range(QB):
        h_ref[q] = h[:, q * LB:(q + 1) * LB]


def _hidden(x, W1, b1):
    return pl.pallas_call(
        _mlp_body,
        grid=(NPAD // BLK,),
        in_specs=[
            pl.BlockSpec((BLK, D), lambda i: (i, 0)),
            pl.BlockSpec((D, H), lambda i: (0, 0)),
            pl.BlockSpec((1, H), lambda i: (0, 0)),
        ],
        out_specs=pl.BlockSpec((QB, BLK, LB), lambda i: (0, i, 0)),
        out_shape=jax.ShapeDtypeStruct((QB, NPAD, LB), jnp.float32),
    )(x, W1, b1.reshape(1, H))


def _segsum_body(h_hbm, src_hbm, dst_hbm, zeros_hbm, out_hbm,
                 src_v, dst_v, buf, acc_sh, sem):
    cid = lax.axis_index("c")
    sid = lax.axis_index("s")

    pltpu.sync_copy(src_hbm.at[sid], src_v)
    pltpu.sync_copy(dst_hbm.at[sid], dst_v)

    for k in range(QB // NC):
        q = cid * (QB // NC) + k

        pltpu.sync_copy(zeros_hbm,
                        acc_sh.at[pl.ds(sid * SLICE, SLICE)])
        plsc.subcore_barrier()

        def step(j, carry):
            pltpu.async_copy(
                h_hbm.at[q].at[src_v.at[j]], buf, sem).wait()
            pltpu.sync_copy(buf, acc_sh.at[dst_v.at[j]], add=True)
            return carry

        lax.fori_loop(0, CHE, step, 0)
        plsc.subcore_barrier()

        pltpu.sync_copy(acc_sh.at[pl.ds(sid * SLICE, SLICE)],
                        out_hbm.at[q, pl.ds(sid * SLICE, SLICE)])
        plsc.subcore_barrier()


@functools.cache
def _segsum():
    return pl.kernel(
        _segsum_body,
        out_type=jax.ShapeDtypeStruct((QB, NPAD, LB), jnp.float32),
        mesh=plsc.VectorSubcoreMesh(core_axis_name="c", subcore_axis_name="s",
                                    num_cores=NC, num_subcores=NS),
        scratch_types=(
            [pltpu.VMEM((CHE, LANES), jnp.int32),
             pltpu.VMEM((CHE, LANES), jnp.int32)]
            + [pltpu.VMEM((LANES, LB), jnp.float32)]
            + [pltpu.VMEM_SHARED((NPAD, LB), jnp.float32)]
            + [pltpu.SemaphoreType.DMA]
        ),
    )


def _pred_body(h_ref, a_ref, w2_ref, b2_ref, g_ref,
               pred_ref, act_ref, lp_ref, scratch):
    i = pl.program_id(0)
    a = jnp.concatenate([h_ref[q] + a_ref[q] for q in range(QB)], axis=1)
    pred_blk = jnp.dot(a, w2_ref[...],
                       preferred_element_type=jnp.float32) + b2_ref[0, 0]
    pred_ref[...] = pred_blk
    scratch[pl.ds(i * (BLK // LANES), BLK // LANES), :] = (
        pred_blk.reshape(BLK // LANES, LANES))

    @pl.when(i == NPAD // BLK - 1)
    def _():
        pred = scratch[...]
        row = lax.broadcasted_iota(jnp.int32, (ROWS, LANES), 0)
        col = lax.broadcasted_iota(jnp.int32, (ROWS, LANES), 1)
        flat = row * LANES + col
        valid = flat < N
        neg_inf = jnp.float32(-jnp.inf)

        m = jnp.max(jnp.where(valid, pred, neg_inf))
        e = jnp.where(valid, jnp.exp(pred - m), 0.0)
        p = e / jnp.sum(e)

        y = jnp.where(valid, jnp.log(p + 1e-20) + g_ref[...], neg_inf)
        ymax = jnp.max(y)
        action = jnp.min(jnp.where(y == ymax, flat, jnp.int32(2147483647)))
        act_ref[0, 0] = action
        lp_ref[0, 0] = jnp.log(jnp.sum(jnp.where(flat == action, p, 0.0)))


def _pred_sample(h_blk, agg_blk, W2, b2, g80):
    return pl.pallas_call(
        _pred_body,
        grid=(NPAD // BLK,),
        in_specs=[
            pl.BlockSpec((QB, BLK, LB), lambda i: (0, i, 0)),
            pl.BlockSpec((QB, BLK, LB), lambda i: (0, i, 0)),
            pl.BlockSpec((H, 1), lambda i: (0, 0)),
            pl.BlockSpec((1, 1), lambda i: (0, 0)),
            pl.BlockSpec((ROWS, LANES), lambda i: (0, 0)),
        ],
        out_specs=[
            pl.BlockSpec((BLK, 1), lambda i: (i, 0)),
            pl.BlockSpec((1, 1), lambda i: (0, 0), memory_space=pltpu.SMEM),
            pl.BlockSpec((1, 1), lambda i: (0, 0), memory_space=pltpu.SMEM),
        ],
        out_shape=[
            jax.ShapeDtypeStruct((NPAD, 1), jnp.float32),
            jax.ShapeDtypeStruct((1, 1), jnp.int32),
            jax.ShapeDtypeStruct((1, 1), jnp.float32),
        ],
        scratch_shapes=[pltpu.VMEM((ROWS, LANES), jnp.float32)],
    )(h_blk, agg_blk, W2, b2.reshape(1, 1), g80)


def kernel(x, edge_index, batch, W1, b1, W2, b2):
    h_blk = _hidden(x, W1, b1)

    src = jnp.concatenate(
        [edge_index[0], jnp.zeros((EPAD - E,), jnp.int32)]).reshape(NS, CHE, LANES)
    dst = jnp.concatenate(
        [edge_index[1], jnp.full((EPAD - E,), N, jnp.int32)]).reshape(NS, CHE, LANES)
    zeros = jnp.zeros((SLICE, LB), jnp.float32)

    agg_blk = _segsum()(h_blk, src, dst, zeros)

    g = jax.random.gumbel(jax.random.key(42), (N,), jnp.float32)
    g80 = jnp.concatenate([g, jnp.zeros((NPAD - N,), jnp.float32)]
                          ).reshape(ROWS, LANES)

    pred_pad, act, lp = _pred_sample(h_blk, agg_blk, W2, b2, g80)
    return (pred_pad[:N], act.reshape(()), lp.reshape(()))
